# Initial kernel scaffold; baseline (speedup 1.0000x reference)
import jax
import jax.numpy as jnp
from jax.experimental import pallas as pl


def kernel(x, edge_index, raw_weights, W_self1, W_neigh1, b1, W_self2, W_neigh2, b2):
    raise NotImplementedError("write your pallas kernel here")


# speedup vs baseline: 3.4172x; 3.4172x over previous
import functools

import jax
import jax.numpy as jnp
from jax import lax
from jax.experimental import pallas as pl
from jax.experimental.pallas import tpu as pltpu
from jax.experimental.pallas import tpu_sc as plsc

N = 10000
D = 128
E = 320000

NC = 2
NS = 16
NW = NC * NS
CHUNK = 128
EPW_REAL = E // NW
NCHUNK = -(-EPW_REAL // CHUNK)
NCHUNK += NCHUNK % 2
EPW = NCHUNK * CHUNK
NPAD = 10240
RPS = NPAD // NS


def _sc_agg_body(with_deg, refs):
    if with_deg:
        (table, src_hbm, dst_hbm, acc_out, deg_out,
         acc_sh, deg_sh, src_cur0, src_cur1, dst_cur0, dst_cur1,
         rows0, onesb, zb, gsem, isem) = refs
    else:
        (table, src_hbm, dst_hbm, acc_out,
         acc_sh, src_cur0, src_cur1, dst_cur0, dst_cur1,
         rows0, gsem, isem) = refs

    c = lax.axis_index("c")
    s = lax.axis_index("s")
    wid = s * NC + c
    base = s * RPS

    zero16 = jnp.zeros((16,), jnp.float32)

    def zrow(i, _):
        r = i // (D // 16)
        k = (i % (D // 16)) * 16
        rows0[r, pl.ds(k, 16)] = zero16
        return 0

    lax.fori_loop(0, CHUNK * (D // 16), zrow, 0)

    if with_deg:
        one16 = jnp.ones((16,), jnp.float32)
        for k in range(CHUNK // 16):
            onesb[pl.ds(k * 16, 16)] = one16
        for k in range(RPS // 16):
            zb[pl.ds(k * 16, 16)] = zero16
        pltpu.sync_copy(zb, deg_sh.at[pl.ds(base, RPS)])

    for j in range(RPS // CHUNK):
        pltpu.sync_copy(rows0, acc_sh.at[pl.ds(base + j * CHUNK, CHUNK)])

    plsc.subcore_barrier()

    def chunk_step(j, src_cur, dst_cur, rows):
        pltpu.sync_copy(src_hbm.at[wid, j], src_cur)
        pltpu.sync_copy(dst_hbm.at[wid, j], dst_cur)
        pltpu.async_copy(table.at[src_cur], rows, gsem).wait()
        pltpu.sync_copy(rows, acc_sh.at[dst_cur], add=True)
        if with_deg:
            pltpu.sync_copy(onesb, deg_sh.at[dst_cur], add=True)

    def g_step(g, _):
        chunk_step(2 * g, src_cur0, dst_cur0, rows0)
        chunk_step(2 * g + 1, src_cur1, dst_cur1, rows0)
        return 0

    lax.fori_loop(0, NCHUNK // 2, g_step, 0)

    plsc.subcore_barrier()

    pltpu.sync_copy(acc_sh.at[pl.ds(base, RPS)],
                    acc_out.at[c, pl.ds(base, RPS)])
    if with_deg:
        pltpu.sync_copy(deg_sh.at[pl.ds(base, RPS)],
                        deg_out.at[c, pl.ds(base, RPS)])


def _make_sc_agg(with_deg):
    out_type = [jax.ShapeDtypeStruct((NC, NPAD, D), jnp.float32)]
    scratch = [
        pltpu.VMEM_SHARED((NPAD, D), jnp.float32),
    ]
    if with_deg:
        out_type.append(jax.ShapeDtypeStruct((NC, NPAD), jnp.float32))
        scratch.append(pltpu.VMEM_SHARED((NPAD,), jnp.float32))
    scratch += [
        pltpu.VMEM((CHUNK,), jnp.int32),
        pltpu.VMEM((CHUNK,), jnp.int32),
        pltpu.VMEM((CHUNK,), jnp.int32),
        pltpu.VMEM((CHUNK,), jnp.int32),
        pltpu.VMEM((CHUNK, D), jnp.float32),
    ]
    if with_deg:
        scratch += [
            pltpu.VMEM((CHUNK,), jnp.float32),
            pltpu.VMEM((RPS,), jnp.float32),
        ]
    scratch.append(pltpu.SemaphoreType.DMA)
    scratch.append(pltpu.SemaphoreType.DMA)

    def body(*refs):
        _sc_agg_body(with_deg, refs)

    return pl.kernel(
        body,
        out_type=out_type,
        mesh=plsc.VectorSubcoreMesh(core_axis_name="c", subcore_axis_name="s"),
        scratch_types=scratch,
        name="sc_seg_sum_deg" if with_deg else "sc_seg_sum",
    )


_sc_agg_deg = _make_sc_agg(True)
_sc_agg = _make_sc_agg(False)


def _tc_weight_body(x_ref, rw_ref, o_ref):
    rw = rw_ref[0, :]
    w = jnp.exp(rw - jnp.max(rw))
    w = w / jnp.sum(w)
    o_ref[...] = x_ref[...] * w[None, :]


def _tc_weight(x, rw):
    return pl.pallas_call(
        _tc_weight_body,
        out_shape=jax.ShapeDtypeStruct((N, D), jnp.float32),
    )(x, rw)


def _tc_sage_body(h_ref, acc_ref, deg_ref, ws_ref, wn_ref, b_ref, o_ref):
    agg = acc_ref[0, :N, :] + acc_ref[1, :N, :]
    deg = deg_ref[:N, 0:1] + deg_ref[:N, 1:2]
    mean = agg / jnp.maximum(deg, 1.0)
    h = h_ref[...]
    dn = (((1,), (1,)), ((), ()))
    hs = lax.dot_general(h, ws_ref[...], dn, preferred_element_type=jnp.float32)
    hn = lax.dot_general(mean, wn_ref[...], dn, preferred_element_type=jnp.float32)
    o_ref[...] = jnp.maximum(hs + hn + b_ref[0, :][None, :], 0.0) + h


def _tc_sage(h, acc, deg, ws, wn, b):
    return pl.pallas_call(
        _tc_sage_body,
        out_shape=jax.ShapeDtypeStruct((N, D), jnp.float32),
    )(h, acc, deg, ws, wn, b)


def kernel(x, edge_index, raw_weights, W_self1, W_neigh1, b1,
           W_self2, W_neigh2, b2):
    src = edge_index[0].reshape(NW, EPW_REAL)
    dst = edge_index[1].reshape(NW, EPW_REAL)
    npad = EPW - EPW_REAL
    src_i = jnp.concatenate(
        [src, jnp.zeros((NW, npad), jnp.int32)], axis=1).reshape(
            NW, NCHUNK, CHUNK)
    dst_i = jnp.concatenate(
        [dst, jnp.full((NW, npad), N, jnp.int32)], axis=1).reshape(
            NW, NCHUNK, CHUNK)

    rw = raw_weights.reshape(1, D)
    wx = _tc_weight(x, rw)
    acc1, deg = _sc_agg_deg(wx, src_i, dst_i)
    deg_t = deg.T
    out1 = _tc_sage(wx, acc1, deg_t, W_self1, W_neigh1, b1.reshape(1, D))
    (acc2,) = _sc_agg(out1, src_i, dst_i)
    out2 = _tc_sage(out1, acc2, deg_t, W_self2, W_neigh2, b2.reshape(1, D))
    return out2


# speedup vs baseline: 4.4775x; 1.3103x over previous
import functools

import jax
import jax.numpy as jnp
from jax import lax
from jax.experimental import pallas as pl
from jax.experimental.pallas import tpu as pltpu
from jax.experimental.pallas import tpu_sc as plsc

N = 10000
D = 128
E = 320000

NC = 2
NS = 16
NW = NC * NS
CHUNK = 128
EPW_REAL = E // NW
NCHUNK = -(-EPW_REAL // CHUNK)
NCHUNK += NCHUNK % 2
EPW = NCHUNK * CHUNK
NPAD = 10240
RPS = NPAD // NS


def _sc_agg_body(with_deg, refs):
    if with_deg:
        (table, pk_hbm, acc_out, deg_out,
         acc_sh, deg_sh, pk_idx, src_cur0, src_cur1, dst_cur0, dst_cur1,
         rows0, rows1, onesb, zb, gsem0, gsem1, isem) = refs
    else:
        (table, pk_hbm, acc_out,
         acc_sh, pk_idx, src_cur0, src_cur1, dst_cur0, dst_cur1,
         rows0, rows1, gsem0, gsem1, isem) = refs

    c = lax.axis_index("c")
    s = lax.axis_index("s")
    wid = s * NC + c
    base = s * RPS

    zero16 = jnp.zeros((16,), jnp.float32)

    idx_cp = pltpu.async_copy(pk_hbm.at[wid], pk_idx, isem)

    def zrow(i, _):
        r = i // (D // 16)
        k = (i % (D // 16)) * 16
        rows0[r, pl.ds(k, 16)] = zero16
        return 0

    lax.fori_loop(0, CHUNK * (D // 16), zrow, 0)

    if with_deg:
        one16 = jnp.ones((16,), jnp.float32)
        for k in range(CHUNK // 16):
            onesb[pl.ds(k * 16, 16)] = one16
        for k in range(RPS // 16):
            zb[pl.ds(k * 16, 16)] = zero16
        pltpu.sync_copy(zb, deg_sh.at[pl.ds(base, RPS)])

    for j in range(RPS // CHUNK):
        pltpu.sync_copy(rows0, acc_sh.at[pl.ds(base + j * CHUNK, CHUNK)])

    idx_cp.wait()
    plsc.subcore_barrier()

    def unpack(j, src_cur, dst_cur):
        for k in range(CHUNK // 16):
            v = pk_idx[j, pl.ds(k * 16, 16)]
            src_cur[pl.ds(k * 16, 16)] = lax.shift_right_logical(v, 14)
            dst_cur[pl.ds(k * 16, 16)] = lax.bitwise_and(v, 16383)

    def start_gather(src_cur, rows, sem):
        pltpu.async_copy(table.at[src_cur], rows, sem)

    def drain_gather(src_cur, rows, sem):
        pltpu.make_async_copy(table.at[src_cur], rows, sem).wait()

    def scatter(dst_cur, rows):
        pltpu.sync_copy(rows, acc_sh.at[dst_cur], add=True)
        if with_deg:
            pltpu.sync_copy(onesb, deg_sh.at[dst_cur], add=True)

    unpack(0, src_cur0, dst_cur0)
    start_gather(src_cur0, rows0, gsem0)
    unpack(1, src_cur1, dst_cur1)
    start_gather(src_cur1, rows1, gsem1)

    def g_step(g, _):
        j = 2 * g
        drain_gather(src_cur0, rows0, gsem0)
        scatter(dst_cur0, rows0)
        unpack(j + 2, src_cur0, dst_cur0)
        start_gather(src_cur0, rows0, gsem0)
        drain_gather(src_cur1, rows1, gsem1)
        scatter(dst_cur1, rows1)
        unpack(j + 3, src_cur1, dst_cur1)
        start_gather(src_cur1, rows1, gsem1)
        return 0

    lax.fori_loop(0, NCHUNK // 2 - 1, g_step, 0)

    drain_gather(src_cur0, rows0, gsem0)
    scatter(dst_cur0, rows0)
    drain_gather(src_cur1, rows1, gsem1)
    scatter(dst_cur1, rows1)

    plsc.subcore_barrier()

    pltpu.sync_copy(acc_sh.at[pl.ds(base, RPS)],
                    acc_out.at[c, pl.ds(base, RPS)])
    if with_deg:
        pltpu.sync_copy(deg_sh.at[pl.ds(base, RPS)],
                        deg_out.at[c, pl.ds(base, RPS)])


def _make_sc_agg(with_deg):
    out_type = [jax.ShapeDtypeStruct((NC, NPAD, D), jnp.float32)]
    scratch = [
        pltpu.VMEM_SHARED((NPAD, D), jnp.float32),
    ]
    if with_deg:
        out_type.append(jax.ShapeDtypeStruct((NC, NPAD), jnp.float32))
        scratch.append(pltpu.VMEM_SHARED((NPAD,), jnp.float32))
    scratch += [
        pltpu.VMEM((NCHUNK, CHUNK), jnp.int32),
        pltpu.VMEM((CHUNK,), jnp.int32),
        pltpu.VMEM((CHUNK,), jnp.int32),
        pltpu.VMEM((CHUNK,), jnp.int32),
        pltpu.VMEM((CHUNK,), jnp.int32),
        pltpu.VMEM((CHUNK, D), jnp.float32),
        pltpu.VMEM((CHUNK, D), jnp.float32),
    ]
    if with_deg:
        scratch += [
            pltpu.VMEM((CHUNK,), jnp.float32),
            pltpu.VMEM((RPS,), jnp.float32),
        ]
    scratch.append(pltpu.SemaphoreType.DMA)
    scratch.append(pltpu.SemaphoreType.DMA)
    scratch.append(pltpu.SemaphoreType.DMA)

    def body(*refs):
        _sc_agg_body(with_deg, refs)

    return pl.kernel(
        body,
        out_type=out_type,
        mesh=plsc.VectorSubcoreMesh(core_axis_name="c", subcore_axis_name="s"),
        scratch_types=scratch,
        name="sc_seg_sum_deg" if with_deg else "sc_seg_sum",
    )


_sc_agg_deg = _make_sc_agg(True)
_sc_agg = _make_sc_agg(False)


def _tc_weight_body(x_ref, rw_ref, o_ref):
    rw = rw_ref[0, :]
    w = jnp.exp(rw - jnp.max(rw))
    w = w / jnp.sum(w)
    o_ref[...] = x_ref[...] * w[None, :]


def _tc_weight(x, rw):
    return pl.pallas_call(
        _tc_weight_body,
        out_shape=jax.ShapeDtypeStruct((N, D), jnp.float32),
    )(x, rw)


def _tc_sage_body(h_ref, acc_ref, deg_ref, ws_ref, wn_ref, b_ref, o_ref):
    agg = acc_ref[0, :N, :] + acc_ref[1, :N, :]
    deg = deg_ref[:N, 0:1] + deg_ref[:N, 1:2]
    mean = agg / jnp.maximum(deg, 1.0)
    h = h_ref[...]
    dn = (((1,), (1,)), ((), ()))
    hs = lax.dot_general(h, ws_ref[...], dn, preferred_element_type=jnp.float32)
    hn = lax.dot_general(mean, wn_ref[...], dn, preferred_element_type=jnp.float32)
    o_ref[...] = jnp.maximum(hs + hn + b_ref[0, :][None, :], 0.0) + h


def _tc_sage(h, acc, deg, ws, wn, b):
    return pl.pallas_call(
        _tc_sage_body,
        out_shape=jax.ShapeDtypeStruct((N, D), jnp.float32),
    )(h, acc, deg, ws, wn, b)


def kernel(x, edge_index, raw_weights, W_self1, W_neigh1, b1,
           W_self2, W_neigh2, b2):
    packed = edge_index[0] * 16384 + edge_index[1]
    packed = packed.reshape(NW, EPW_REAL)
    npad = EPW - EPW_REAL
    pk_i = jnp.concatenate(
        [packed, jnp.full((NW, npad), N, jnp.int32)], axis=1).reshape(
            NW, NCHUNK, CHUNK)

    rw = raw_weights.reshape(1, D)
    wx = _tc_weight(x, rw)
    acc1, deg = _sc_agg_deg(wx, pk_i)
    deg_t = deg.T
    out1 = _tc_sage(wx, acc1, deg_t, W_self1, W_neigh1, b1.reshape(1, D))
    (acc2,) = _sc_agg(out1, pk_i)
    out2 = _tc_sage(out1, acc2, deg_t, W_self2, W_neigh2, b2.reshape(1, D))
    return out2
